# baseline (speedup 1.0000x reference)
import functools

import jax
import jax.numpy as jnp
from jax import lax
from jax.experimental import pallas as pl
from jax.experimental.pallas import tpu as pltpu
from jax.experimental.pallas import tpu_sc as plsc

_NUM_CAPS = 8
_NITER = 3
_TAU = 1.0
_N_FOLD = 20
_DD = 16


def _caps_mask():
    li = lax.broadcasted_iota(jnp.int32, (128, 128), 0) // _DD
    lj = lax.broadcasted_iota(jnp.int32, (128, 128), 1) // _DD
    return (li == lj).astype(jnp.float32)


def _percap(x, S):
    return lax.dot_general(x, S, (((1,), (0,)), ((), ())),
                           preferred_element_type=jnp.float32)


def _normalize_rows(u, S):
    ss = _percap(u * u, S)
    return u / jnp.maximum(jnp.sqrt(ss), 1e-12)


def _norm_caps(v):
    def kern(v_ref, o_ref):
        o_ref[...] = _normalize_rows(v_ref[...], _caps_mask())
    return pl.pallas_call(
        kern, out_shape=jax.ShapeDtypeStruct(v.shape, jnp.float32))(v)


def _edge_embed(adj, xn):
    e = adj.shape[0]
    eb = 200
    def kern(a_ref, x_ref, o_ref):
        emb = jnp.dot(a_ref[...], x_ref[...],
                      preferred_element_type=jnp.float32)
        o_ref[...] = _normalize_rows(emb, _caps_mask())
    return pl.pallas_call(
        kern,
        grid=(e // eb,),
        in_specs=[
            pl.BlockSpec((eb, adj.shape[1]), lambda i: (i, 0)),
            pl.BlockSpec(xn.shape, lambda i: (0, 0)),
        ],
        out_specs=pl.BlockSpec((eb, 128), lambda i: (i, 0)),
        out_shape=jax.ShapeDtypeStruct((e, 128), jnp.float32),
    )(adj, xn)


def _sc_gather(table, idx):
    B = idx.shape[0]
    D = table.shape[1]
    info = plsc.get_sparse_core_info()
    NW = info.num_cores * info.num_subcores
    CHUNK = 128
    total = B // CHUNK
    per_w = -(-total // NW)
    mesh = plsc.VectorSubcoreMesh(core_axis_name="c", subcore_axis_name="s")

    @functools.partial(
        pl.kernel, mesh=mesh,
        out_type=jax.ShapeDtypeStruct((B, D), jnp.float32),
        scratch_types=[
            pltpu.VMEM((CHUNK,), jnp.int32),
            pltpu.VMEM((CHUNK, D), jnp.float32),
            pltpu.SemaphoreType.DMA,
        ],
    )
    def k(table_hbm, idx_hbm, out_hbm, idx_v, rows_v, sem):
        wid = lax.axis_index("s") * info.num_cores + lax.axis_index("c")

        def body(j, carry):
            t = j * NW + wid

            @pl.when(t < total)
            def _():
                base = t * CHUNK
                pltpu.sync_copy(idx_hbm.at[pl.ds(base, CHUNK)], idx_v)
                pltpu.async_copy(table_hbm.at[idx_v], rows_v, sem).wait()
                pltpu.sync_copy(rows_v, out_hbm.at[pl.ds(base, CHUNK)])
            return carry

        lax.fori_loop(0, per_w, body, 0)

    return k(table, idx)


def _phase(u_init, ids, z, jb, interpret=False):
    T = u_init.shape[0]
    nch_tot, _, MB = ids.shape
    CH = nch_tot // _N_FOLD
    NJB = T // jb

    def kern(ids_ref, z_ref, u0_ref, out_ref, uread, uacc):
        it = pl.program_id(0)
        f = pl.program_id(1)
        c = pl.program_id(2)

        @pl.when((it == 0) & (f == 0) & (c == 0))
        def _():
            uread[...] = u0_ref[...]
            uacc[...] = u0_ref[...]

        S = _caps_mask()
        ids_row = jnp.squeeze(ids_ref[...], axis=0)[0:1, :]
        zc = z_ref[...]

        ug = jnp.zeros((MB, 128), jnp.float32)
        for b in range(NJB):
            row_iota = lax.broadcasted_iota(jnp.int32, (jb, MB), 0) + b * jb
            ohT = (row_iota == ids_row).astype(jnp.float32)
            ub = uread[pl.ds(b * jb, jb), :]
            ug = ug + lax.dot_general(ohT, ub, (((0,), (0,)), ((), ())),
                                      preferred_element_type=jnp.float32)

        logits = _percap(zc * ug, S) / _TAU
        mx = jnp.max(logits, axis=1, keepdims=True)
        ex = jnp.exp(logits - mx)
        den = jnp.sum(ex, axis=1, keepdims=True) / _DD
        msg = zc * (ex / den)

        for b in range(NJB):
            row_iota = lax.broadcasted_iota(jnp.int32, (jb, MB), 0) + b * jb
            ohT = (row_iota == ids_row).astype(jnp.float32)
            upd = lax.dot_general(ohT, msg, (((1,), (0,)), ((), ())),
                                  preferred_element_type=jnp.float32)
            uacc[pl.ds(b * jb, jb), :] = uacc[pl.ds(b * jb, jb), :] + upd

        @pl.when(c == CH - 1)
        def _():
            @pl.when(f == _N_FOLD - 1)
            def _():
                uacc[...] = _normalize_rows(uacc[...], S)
            uread[...] = uacc[...]

        @pl.when((it == _NITER - 1) & (f == _N_FOLD - 1) & (c == CH - 1))
        def _():
            out_ref[...] = uacc[...]

    return pl.pallas_call(
        kern,
        grid=(_NITER, _N_FOLD, CH),
        in_specs=[
            pl.BlockSpec((1, 8, MB), lambda it, f, c: (f * CH + c, 0, 0)),
            pl.BlockSpec((MB, 128), lambda it, f, c: (f * CH + c, 0)),
            pl.BlockSpec((T, 128), lambda it, f, c: (0, 0)),
        ],
        out_specs=pl.BlockSpec((T, 128), lambda it, f, c: (0, 0)),
        out_shape=jax.ShapeDtypeStruct((T, 128), jnp.float32),
        scratch_shapes=[
            pltpu.VMEM((T, 128), jnp.float32),
            pltpu.VMEM((T, 128), jnp.float32),
        ],
        interpret=interpret,
    )(ids, z, u_init)


_MB = 1000


def _chunk_ids(v):
    c = v.reshape(-1, 1, _MB)
    return jnp.broadcast_to(c, (c.shape[0], 8, _MB)).astype(jnp.int32)


def kernel(x, adjacency, edge_node):
    n = x.shape[0]
    e = adjacency.shape[0]

    edge_es = edge_node[0]
    node_es = edge_node[1]
    perm = jnp.argsort(node_es)
    node_ns = node_es[perm]
    edge_ns = edge_es[perm]

    xn = _norm_caps(x)
    u0 = _edge_embed(adjacency, xn)

    z1 = _sc_gather(xn, node_es)
    u_edge = _phase(u0, _chunk_ids(edge_es), z1, jb=e)

    z2 = _sc_gather(u_edge, edge_ns)
    u_node = _phase(xn, _chunk_ids(node_ns), z2, jb=2000)

    return (u_node, u_edge)


# speedup vs baseline: 3.3976x; 3.3976x over previous
import functools

import jax
import jax.numpy as jnp
from jax import lax
from jax.experimental import pallas as pl
from jax.experimental.pallas import tpu as pltpu
from jax.experimental.pallas import tpu_sc as plsc

_NUM_CAPS = 8
_NITER = 3
_TAU = 1.0
_N_FOLD = 20
_DD = 16


def _caps_mask():
    li = lax.broadcasted_iota(jnp.int32, (128, 128), 0) // _DD
    lj = lax.broadcasted_iota(jnp.int32, (128, 128), 1) // _DD
    return (li == lj).astype(jnp.float32)


def _percap(x, S):
    return lax.dot_general(x, S, (((1,), (0,)), ((), ())),
                           preferred_element_type=jnp.float32)


def _normalize_rows(u, S):
    ss = _percap(u * u, S)
    return u / jnp.maximum(jnp.sqrt(ss), 1e-12)


def _norm_caps(v):
    def kern(v_ref, o_ref):
        o_ref[...] = _normalize_rows(v_ref[...], _caps_mask())
    return pl.pallas_call(
        kern, out_shape=jax.ShapeDtypeStruct(v.shape, jnp.float32))(v)


def _edge_embed(adj, xn):
    e = adj.shape[0]
    eb = 200
    def kern(a_ref, x_ref, o_ref):
        emb = jnp.dot(a_ref[...], x_ref[...],
                      preferred_element_type=jnp.float32)
        o_ref[...] = _normalize_rows(emb, _caps_mask())
    return pl.pallas_call(
        kern,
        grid=(e // eb,),
        in_specs=[
            pl.BlockSpec((eb, adj.shape[1]), lambda i: (i, 0)),
            pl.BlockSpec(xn.shape, lambda i: (0, 0)),
        ],
        out_specs=pl.BlockSpec((eb, 128), lambda i: (i, 0)),
        out_shape=jax.ShapeDtypeStruct((e, 128), jnp.float32),
    )(adj, xn)


def _sc_gather(table, idx):
    B = idx.shape[0]
    D = table.shape[1]
    info = plsc.get_sparse_core_info()
    NW = info.num_cores * info.num_subcores
    CHUNK = 128
    total = B // CHUNK
    per_w = -(-total // NW)
    mesh = plsc.VectorSubcoreMesh(core_axis_name="c", subcore_axis_name="s")

    @functools.partial(
        pl.kernel, mesh=mesh,
        out_type=jax.ShapeDtypeStruct((B, D), jnp.float32),
        scratch_types=[
            pltpu.VMEM((CHUNK,), jnp.int32),
            pltpu.VMEM((CHUNK, D), jnp.float32),
            pltpu.SemaphoreType.DMA,
        ],
    )
    def k(table_hbm, idx_hbm, out_hbm, idx_v, rows_v, sem):
        wid = lax.axis_index("s") * info.num_cores + lax.axis_index("c")

        def body(j, carry):
            t = j * NW + wid

            @pl.when(t < total)
            def _():
                base = t * CHUNK
                pltpu.sync_copy(idx_hbm.at[pl.ds(base, CHUNK)], idx_v)
                pltpu.async_copy(table_hbm.at[idx_v], rows_v, sem).wait()
                pltpu.sync_copy(rows_v, out_hbm.at[pl.ds(base, CHUNK)])
            return carry

        lax.fori_loop(0, per_w, body, 0)

    return k(table, idx)


def _phase(u_init, ids, bounds, z, jb, interpret=False):
    T = u_init.shape[0]
    nch_tot, _, MB = ids.shape
    CH = nch_tot // _N_FOLD
    NJB = T // jb

    def kern(ids_ref, b_ref, z_ref, u0_ref, out_ref, uread, uacc, ug_ref):
        it = pl.program_id(0)
        f = pl.program_id(1)
        c = pl.program_id(2)
        ci = f * CH + c
        lo = b_ref[ci, 0]
        hi = b_ref[ci, 1]

        @pl.when((it == 0) & (f == 0) & (c == 0))
        def _():
            uread[...] = u0_ref[...]
            uacc[...] = u0_ref[...]

        S = _caps_mask()
        ids_row = jnp.squeeze(ids_ref[...], axis=0)[0:1, :]
        zc = z_ref[...]

        ug_ref[...] = jnp.zeros((MB, 128), jnp.float32)
        for b in range(NJB):
            @pl.when((lo < (b + 1) * jb) & (hi >= b * jb))
            def _():
                row_iota = (lax.broadcasted_iota(jnp.int32, (jb, MB), 0)
                            + b * jb)
                ohT = (row_iota == ids_row).astype(jnp.float32)
                ub = uread[pl.ds(b * jb, jb), :]
                ug_ref[...] = ug_ref[...] + lax.dot_general(
                    ohT, ub, (((0,), (0,)), ((), ())),
                    preferred_element_type=jnp.float32)
        ug = ug_ref[...]

        logits = _percap(zc * ug, S) / _TAU
        mx = jnp.max(logits, axis=1, keepdims=True)
        ex = jnp.exp(logits - mx)
        den = jnp.sum(ex, axis=1, keepdims=True) / _DD
        msg = zc * (ex / den)

        for b in range(NJB):
            @pl.when((lo < (b + 1) * jb) & (hi >= b * jb))
            def _():
                row_iota = (lax.broadcasted_iota(jnp.int32, (jb, MB), 0)
                            + b * jb)
                ohT = (row_iota == ids_row).astype(jnp.float32)
                upd = lax.dot_general(ohT, msg, (((1,), (0,)), ((), ())),
                                      preferred_element_type=jnp.float32)
                uacc[pl.ds(b * jb, jb), :] = uacc[pl.ds(b * jb, jb), :] + upd

        @pl.when(c == CH - 1)
        def _():
            @pl.when(f == _N_FOLD - 1)
            def _():
                uacc[...] = _normalize_rows(uacc[...], S)
            uread[...] = uacc[...]

        @pl.when((it == _NITER - 1) & (f == _N_FOLD - 1) & (c == CH - 1))
        def _():
            out_ref[...] = uacc[...]

    return pl.pallas_call(
        kern,
        grid=(_NITER, _N_FOLD, CH),
        in_specs=[
            pl.BlockSpec((1, 8, MB), lambda it, f, c: (f * CH + c, 0, 0)),
            pl.BlockSpec(memory_space=pltpu.SMEM),
            pl.BlockSpec((MB, 128), lambda it, f, c: (f * CH + c, 0)),
            pl.BlockSpec((T, 128), lambda it, f, c: (0, 0)),
        ],
        out_specs=pl.BlockSpec((T, 128), lambda it, f, c: (0, 0)),
        out_shape=jax.ShapeDtypeStruct((T, 128), jnp.float32),
        scratch_shapes=[
            pltpu.VMEM((T, 128), jnp.float32),
            pltpu.VMEM((T, 128), jnp.float32),
            pltpu.VMEM((MB, 128), jnp.float32),
        ],
        interpret=interpret,
    )(ids, bounds, z, u_init)


_MB = 1000


def _chunk_ids(v):
    c = v.reshape(-1, 1, _MB)
    rep = jnp.broadcast_to(c, (c.shape[0], 8, _MB)).astype(jnp.int32)
    c2 = c[:, 0, :]
    bounds = jnp.stack([jnp.min(c2, axis=1), jnp.max(c2, axis=1)], axis=1)
    return rep, bounds.astype(jnp.int32)


def kernel(x, adjacency, edge_node):
    n = x.shape[0]
    e = adjacency.shape[0]

    edge_es = edge_node[0]
    node_es = edge_node[1]
    perm = jnp.argsort(node_es)
    node_ns = node_es[perm]
    edge_ns = edge_es[perm]

    xn = _norm_caps(x)
    u0 = _edge_embed(adjacency, xn)

    z1 = _sc_gather(xn, node_es)
    ids1, bnd1 = _chunk_ids(edge_es)
    u_edge = _phase(u0, ids1, bnd1, z1, jb=e)

    z2 = _sc_gather(u_edge, edge_ns)
    ids2, bnd2 = _chunk_ids(node_ns)
    u_node = _phase(xn, ids2, bnd2, z2, jb=2000)

    return (u_node, u_edge)


# speedup vs baseline: 5.0749x; 1.4937x over previous
import functools

import jax
import jax.numpy as jnp
from jax import lax
from jax.experimental import pallas as pl
from jax.experimental.pallas import tpu as pltpu
from jax.experimental.pallas import tpu_sc as plsc

_NUM_CAPS = 8
_NITER = 3
_TAU = 1.0
_N_FOLD = 20
_DD = 16


def _caps_mask():
    li = lax.broadcasted_iota(jnp.int32, (128, 128), 0) // _DD
    lj = lax.broadcasted_iota(jnp.int32, (128, 128), 1) // _DD
    return (li == lj).astype(jnp.float32)


def _percap(x, S):
    return lax.dot_general(x, S, (((1,), (0,)), ((), ())),
                           preferred_element_type=jnp.float32)


def _normalize_rows(u, S):
    ss = _percap(u * u, S)
    return u / jnp.maximum(jnp.sqrt(ss), 1e-12)


def _norm_caps(v):
    def kern(v_ref, o_ref):
        o_ref[...] = _normalize_rows(v_ref[...], _caps_mask())
    return pl.pallas_call(
        kern, out_shape=jax.ShapeDtypeStruct(v.shape, jnp.float32))(v)


def _edge_embed(adj, xn):
    e = adj.shape[0]
    eb = 200
    def kern(a_ref, x_ref, o_ref):
        emb = jnp.dot(a_ref[...], x_ref[...],
                      preferred_element_type=jnp.float32)
        o_ref[...] = _normalize_rows(emb, _caps_mask())
    return pl.pallas_call(
        kern,
        grid=(e // eb,),
        in_specs=[
            pl.BlockSpec((eb, adj.shape[1]), lambda i: (i, 0)),
            pl.BlockSpec(xn.shape, lambda i: (0, 0)),
        ],
        out_specs=pl.BlockSpec((eb, 128), lambda i: (i, 0)),
        out_shape=jax.ShapeDtypeStruct((e, 128), jnp.float32),
    )(adj, xn)


def _sc_gather(table, idx):
    B = idx.shape[0]
    D = table.shape[1]
    info = plsc.get_sparse_core_info()
    NW = info.num_cores * info.num_subcores
    CHUNK = 128
    total = B // CHUNK
    per_w = -(-total // NW)
    mesh = plsc.VectorSubcoreMesh(core_axis_name="c", subcore_axis_name="s")

    @functools.partial(
        pl.kernel, mesh=mesh,
        out_type=jax.ShapeDtypeStruct((B, D), jnp.float32),
        scratch_types=[
            pltpu.VMEM((CHUNK,), jnp.int32),
            pltpu.VMEM((CHUNK, D), jnp.float32),
            pltpu.SemaphoreType.DMA,
        ],
    )
    def k(table_hbm, idx_hbm, out_hbm, idx_v, rows_v, sem):
        wid = lax.axis_index("s") * info.num_cores + lax.axis_index("c")

        def body(j, carry):
            t = j * NW + wid

            @pl.when(t < total)
            def _():
                base = t * CHUNK
                pltpu.sync_copy(idx_hbm.at[pl.ds(base, CHUNK)], idx_v)
                pltpu.async_copy(table_hbm.at[idx_v], rows_v, sem).wait()
                pltpu.sync_copy(rows_v, out_hbm.at[pl.ds(base, CHUNK)])
            return carry

        lax.fori_loop(0, per_w, body, 0)

    return k(table, idx)


def _phase(u_init, ids, bounds, z, jb, interpret=False):
    T = u_init.shape[0]
    nch_tot, _, MB = ids.shape
    CH = nch_tot // _N_FOLD
    NJB = T // jb

    def kern(ids_ref, b_ref, z_ref, u0_ref, out_ref, uread, uacc, ug_ref):
        it = pl.program_id(0)
        f = pl.program_id(1)
        c = pl.program_id(2)
        ci = f * CH + c
        lo = b_ref[ci, 0]
        hi = b_ref[ci, 1]

        @pl.when((it == 0) & (f == 0) & (c == 0))
        def _():
            uread[...] = u0_ref[...]
            uacc[...] = u0_ref[...]

        S = _caps_mask()
        ids_row = jnp.squeeze(ids_ref[...], axis=0)[0:1, :]
        zc = z_ref[...]

        ug_ref[...] = jnp.zeros((MB, 128), jnp.float32)
        for b in range(NJB):
            @pl.when((lo < (b + 1) * jb) & (hi >= b * jb))
            def _():
                row_iota = (lax.broadcasted_iota(jnp.int32, (jb, MB), 0)
                            + b * jb)
                ohT = (row_iota == ids_row).astype(jnp.float32)
                ub = uread[pl.ds(b * jb, jb), :]
                ug_ref[...] = ug_ref[...] + lax.dot_general(
                    ohT, ub, (((0,), (0,)), ((), ())),
                    preferred_element_type=jnp.float32)
        ug = ug_ref[...]

        logits = _percap(zc * ug, S) / _TAU
        mx = jnp.max(logits, axis=1, keepdims=True)
        ex = jnp.exp(logits - mx)
        den = jnp.sum(ex, axis=1, keepdims=True) / _DD
        msg = zc * (ex / den)

        for b in range(NJB):
            @pl.when((lo < (b + 1) * jb) & (hi >= b * jb))
            def _():
                row_iota = (lax.broadcasted_iota(jnp.int32, (jb, MB), 0)
                            + b * jb)
                ohT = (row_iota == ids_row).astype(jnp.float32)
                upd = lax.dot_general(ohT, msg, (((1,), (0,)), ((), ())),
                                      preferred_element_type=jnp.float32)
                uacc[pl.ds(b * jb, jb), :] = uacc[pl.ds(b * jb, jb), :] + upd

        @pl.when(c == CH - 1)
        def _():
            @pl.when(f == _N_FOLD - 1)
            def _():
                uacc[...] = _normalize_rows(uacc[...], S)
            uread[...] = uacc[...]

        @pl.when((it == _NITER - 1) & (f == _N_FOLD - 1) & (c == CH - 1))
        def _():
            out_ref[...] = uacc[...]

    return pl.pallas_call(
        kern,
        grid=(_NITER, _N_FOLD, CH),
        in_specs=[
            pl.BlockSpec((1, 8, MB), lambda it, f, c: (f * CH + c, 0, 0)),
            pl.BlockSpec(memory_space=pltpu.SMEM),
            pl.BlockSpec((MB, 128), lambda it, f, c: (f * CH + c, 0)),
            pl.BlockSpec((T, 128), lambda it, f, c: (0, 0)),
        ],
        out_specs=pl.BlockSpec((T, 128), lambda it, f, c: (0, 0)),
        out_shape=jax.ShapeDtypeStruct((T, 128), jnp.float32),
        scratch_shapes=[
            pltpu.VMEM((T, 128), jnp.float32),
            pltpu.VMEM((T, 128), jnp.float32),
            pltpu.VMEM((MB, 128), jnp.float32),
        ],
        interpret=interpret,
    )(ids, bounds, z, u_init)


_MB = 1000


def _chunk_ids(v):
    c = v.reshape(-1, 1, _MB)
    rep = jnp.broadcast_to(c, (c.shape[0], 8, _MB)).astype(jnp.int32)
    c2 = c[:, 0, :]
    bounds = jnp.stack([jnp.min(c2, axis=1), jnp.max(c2, axis=1)], axis=1)
    return rep, bounds.astype(jnp.int32)


def kernel(x, adjacency, edge_node):
    n = x.shape[0]
    e = adjacency.shape[0]

    edge_es = edge_node[0]
    node_es = edge_node[1]
    perm = jnp.argsort(node_es)
    node_ns = node_es[perm]
    edge_ns = edge_es[perm]

    xn = _norm_caps(x)
    u0 = _edge_embed(adjacency, xn)

    ef = edge_es.reshape(_N_FOLD, -1)
    pf = jnp.argsort(ef, axis=1)
    edge_es_s = jnp.take_along_axis(ef, pf, axis=1).reshape(-1)
    node_es_s = jnp.take_along_axis(
        node_es.reshape(_N_FOLD, -1), pf, axis=1).reshape(-1)

    z1 = _sc_gather(xn, node_es_s)
    ids1, bnd1 = _chunk_ids(edge_es_s)
    u_edge = _phase(u0, ids1, bnd1, z1, jb=400)

    z2 = _sc_gather(u_edge, edge_ns)
    ids2, bnd2 = _chunk_ids(node_ns)
    u_node = _phase(xn, ids2, bnd2, z2, jb=500)

    return (u_node, u_edge)
